# baseline (speedup 1.0000x reference)
import jax
import jax.numpy as jnp
from jax.experimental import pallas as pl

E = 320000
C = 128
BLK = 8000


def _block_kernel(linv_ref, adj_ref, w_ref, b_ref, out_ref):
    h = jnp.maximum(adj_ref[...] * linv_ref[...], 0.0)
    out_ref[...] = jnp.sum(h * w_ref[...], axis=1, keepdims=True) + b_ref[0, 0]


def kernel(x_e, Linv, adjacency, W, b):
    del x_e
    grid = (E // BLK,)
    out = pl.pallas_call(
        _block_kernel,
        grid=grid,
        in_specs=[
            pl.BlockSpec((BLK, C), lambda i: (i, 0)),
            pl.BlockSpec((BLK, C), lambda i: (i, 0)),
            pl.BlockSpec((1, C), lambda i: (0, 0)),
            pl.BlockSpec((1, 1), lambda i: (0, 0)),
        ],
        out_specs=pl.BlockSpec((BLK, 1), lambda i: (i, 0)),
        out_shape=jax.ShapeDtypeStruct((E, 1), jnp.float32),
    )(Linv, adjacency, W, b.reshape(1, 1))
    return out


# speedup vs baseline: 1.0066x; 1.0066x over previous
import jax
import jax.numpy as jnp
from jax.experimental import pallas as pl

E = 320000
C = 128
BLK = 8000


def _block_kernel(linv_ref, adj_ref, wt_ref, b_ref, out_ref):
    h = jnp.maximum(adj_ref[...] * linv_ref[...], 0.0)
    out_ref[...] = jax.lax.dot_general(
        h, wt_ref[...], (((1,), (0,)), ((), ())),
        preferred_element_type=jnp.float32,
    ) + b_ref[0, 0]


def kernel(x_e, Linv, adjacency, W, b):
    del x_e
    grid = (E // BLK,)
    out = pl.pallas_call(
        _block_kernel,
        grid=grid,
        in_specs=[
            pl.BlockSpec((BLK, C), lambda i: (i, 0)),
            pl.BlockSpec((BLK, C), lambda i: (i, 0)),
            pl.BlockSpec((C, 1), lambda i: (0, 0)),
            pl.BlockSpec((1, 1), lambda i: (0, 0)),
        ],
        out_specs=pl.BlockSpec((BLK, 1), lambda i: (i, 0)),
        out_shape=jax.ShapeDtypeStruct((E, 1), jnp.float32),
    )(Linv, adjacency, W.T, b.reshape(1, 1))
    return out


# speedup vs baseline: 1.0075x; 1.0009x over previous
import jax
import jax.numpy as jnp
from jax.experimental import pallas as pl

E = 320000
C = 128
NSUB = 8
CH = 2000
STEP = NSUB * CH


def _block_kernel(*refs):
    linv_refs = refs[:NSUB]
    adj_refs = refs[NSUB:2 * NSUB]
    wt_ref = refs[2 * NSUB]
    b_ref = refs[2 * NSUB + 1]
    out_ref = refs[2 * NSUB + 2]
    wt = wt_ref[...]
    bias = b_ref[0, 0]
    for j in range(NSUB):
        h = jnp.maximum(adj_refs[j][...] * linv_refs[j][...], 0.0)
        out_ref[pl.ds(j * CH, CH), :] = jax.lax.dot_general(
            h, wt, (((1,), (0,)), ((), ())),
            preferred_element_type=jnp.float32,
        ) + bias


def kernel(x_e, Linv, adjacency, W, b):
    del x_e
    grid = (E // STEP,)
    sub_specs = [
        pl.BlockSpec((CH, C), lambda i, j=j: (i * NSUB + j, 0))
        for j in range(NSUB)
    ]
    out = pl.pallas_call(
        _block_kernel,
        grid=grid,
        in_specs=sub_specs + sub_specs + [
            pl.BlockSpec((C, 1), lambda i: (0, 0)),
            pl.BlockSpec((1, 1), lambda i: (0, 0)),
        ],
        out_specs=pl.BlockSpec((STEP, 1), lambda i: (i, 0)),
        out_shape=jax.ShapeDtypeStruct((E, 1), jnp.float32),
    )(*([Linv] * NSUB + [adjacency] * NSUB + [W.T, b.reshape(1, 1)]))
    return out


# speedup vs baseline: 2.2831x; 2.2662x over previous
import jax
import jax.numpy as jnp
from jax.experimental import pallas as pl
from jax.experimental.pallas import tpu as pltpu

E = 320000
C = 128
CH = 2000
NCHUNK = E // CH
NBUF = 8
NOUTER = NCHUNK // NBUF


def _in_copy(hbm_ref, buf_ref, sem_ref, i, s):
    return pltpu.make_async_copy(
        hbm_ref.at[pl.ds(i * CH, CH), :], buf_ref.at[s], sem_ref.at[s])


def _out_copy(out_hbm, outbuf, sem_ref, i, s):
    return pltpu.make_async_copy(
        outbuf.at[s], out_hbm.at[pl.ds(i, 1), :], sem_ref.at[s])


def _body(linv_hbm, adj_hbm, w_ref, b_ref, out_hbm,
          linv_buf, adj_buf, outbuf, sem_l, sem_a, sem_o):
    w = w_ref[...]
    bias = b_ref[0, 0]

    for s in range(NBUF):
        _in_copy(linv_hbm, linv_buf, sem_l, s, s).start()
        _in_copy(adj_hbm, adj_buf, sem_a, s, s).start()

    def outer(g, carry):
        for s in range(NBUF):
            i = g * NBUF + s
            _in_copy(linv_hbm, linv_buf, sem_l, i, s).wait()
            _in_copy(adj_hbm, adj_buf, sem_a, i, s).wait()

            h = jnp.maximum(adj_buf[s] * linv_buf[s], 0.0)
            res = jax.lax.dot_general(
                w, h, (((1,), (1,)), ((), ())),
                preferred_element_type=jnp.float32,
            ) + bias

            @pl.when(g > 0)
            def _wait_out():
                _out_copy(out_hbm, outbuf, sem_o, i - NBUF, s).wait()

            outbuf[s] = res

            @pl.when(i + NBUF < NCHUNK)
            def _next_in():
                _in_copy(linv_hbm, linv_buf, sem_l, i + NBUF, s).start()
                _in_copy(adj_hbm, adj_buf, sem_a, i + NBUF, s).start()

            _out_copy(out_hbm, outbuf, sem_o, i, s).start()
        return carry

    jax.lax.fori_loop(0, NOUTER, outer, 0)

    for s in range(NBUF):
        _out_copy(out_hbm, outbuf, sem_o, NCHUNK - NBUF + s, s).wait()


def kernel(x_e, Linv, adjacency, W, b):
    del x_e
    out = pl.pallas_call(
        _body,
        in_specs=[
            pl.BlockSpec(memory_space=pltpu.MemorySpace.HBM),
            pl.BlockSpec(memory_space=pltpu.MemorySpace.HBM),
            pl.BlockSpec(memory_space=pltpu.MemorySpace.VMEM),
            pl.BlockSpec(memory_space=pltpu.MemorySpace.VMEM),
        ],
        out_specs=pl.BlockSpec(memory_space=pltpu.MemorySpace.HBM),
        out_shape=jax.ShapeDtypeStruct((NCHUNK, CH), jnp.float32),
        scratch_shapes=[
            pltpu.VMEM((NBUF, CH, C), jnp.float32),
            pltpu.VMEM((NBUF, CH, C), jnp.float32),
            pltpu.VMEM((NBUF, 1, CH), jnp.float32),
            pltpu.SemaphoreType.DMA((NBUF,)),
            pltpu.SemaphoreType.DMA((NBUF,)),
            pltpu.SemaphoreType.DMA((NBUF,)),
        ],
    )(Linv, adjacency, W, b.reshape(1, 1))
    return out.reshape(E, 1)
